# baseline (speedup 1.0000x reference)
import jax
import jax.numpy as jnp
from jax.experimental import pallas as pl
from jax.experimental.pallas import tpu as pltpu

N = 10000
D = 64
B = 2048
ALPHA = 0.5
RB = 400
NA = N // RB
GB = 256
NB = B // GB
NBUF = 3
HR = RB // 2
HALF = N // 2


def _body(id_ref, adj_ref, init_ref, x0_ref, iga_ref, igb_ref, w1_ref, w2_ref, td_ref,
          ft_ref, p_ref, whh_ref, wf_ref, wt_ref, wp_ref, b_ref, o_ref,
          a_buf, i_buf, s_buf, sem, isem):
    i = pl.program_id(0)
    nsteps = pl.num_programs(0)

    def issue(blk):
        slot = blk % NBUF

        @pl.when(blk < NA)
        def _():
            pltpu.make_async_copy(
                adj_ref.at[pl.ds(blk * HR, HR), :],
                a_buf.at[slot, pl.ds(0, HR), :], sem.at[slot]).start()
            pltpu.make_async_copy(
                adj_ref.at[pl.ds(HALF + blk * HR, HR), :],
                a_buf.at[slot, pl.ds(HR, HR), :], isem.at[slot]).start()

        @pl.when(blk >= NA)
        def _():
            def body(j, _):
                r = id_ref[(blk - NA) * GB + j]
                pltpu.make_async_copy(
                    adj_ref.at[pl.ds(r, 1), :],
                    a_buf.at[slot, pl.ds(j, 1), :], sem.at[slot]).start()
                pltpu.make_async_copy(
                    init_ref.at[pl.ds(r, 1), :],
                    i_buf.at[slot, pl.ds(j, 1), :], isem.at[slot]).start()
                return 0
            jax.lax.fori_loop(0, GB, body, 0, unroll=8)

    @pl.when(i == 0)
    def _():
        issue(0)
        issue(1)

    @pl.when(i + 2 < nsteps)
    def _():
        issue(i + 2)

    slot = i % NBUF

    @pl.when(i < NA)
    def _():
        pltpu.make_async_copy(
            adj_ref.at[pl.ds(0, HR), :], a_buf.at[slot, pl.ds(0, HR), :],
            sem.at[slot]).wait()
        pltpu.make_async_copy(
            adj_ref.at[pl.ds(0, HR), :], a_buf.at[slot, pl.ds(0, HR), :],
            isem.at[slot]).wait()

        @pl.when(i == 0)
        def _():
            s_buf[:, :D] = jnp.dot(x0_ref[...], w1_ref[...],
                                   preferred_element_type=jnp.float32)

        y = jnp.dot(a_buf[slot], s_buf[:, :D],
                    preferred_element_type=jnp.float32)
        ig = jnp.concatenate((iga_ref[...], igb_ref[...]), axis=0)
        x1b = jnp.maximum(ALPHA * y + (1.0 - ALPHA) * ig, 0.0)
        s_buf[pl.ds(i * HR, HR), D:] = jnp.dot(
            x1b[:HR], w2_ref[...], preferred_element_type=jnp.float32)
        s_buf[pl.ds(HALF + i * HR, HR), D:] = jnp.dot(
            x1b[HR:], w2_ref[...], preferred_element_type=jnp.float32)

    @pl.when(i >= NA)
    def _():
        def wbody(j, _):
            pltpu.make_async_copy(
                adj_ref.at[pl.ds(0, 1), :], a_buf.at[slot, pl.ds(0, 1), :],
                sem.at[slot]).wait()
            pltpu.make_async_copy(
                init_ref.at[pl.ds(0, 1), :], i_buf.at[slot, pl.ds(0, 1), :],
                isem.at[slot]).wait()
            return 0
        jax.lax.fori_loop(0, GB, wbody, 0, unroll=8)

        y = jnp.dot(a_buf[slot, :GB, :], s_buf[:, D:],
                    preferred_element_type=jnp.float32)
        x2g = jnp.maximum(ALPHA * y + (1.0 - ALPHA) * i_buf[slot], 0.0)
        const = jnp.dot(p_ref[...], wp_ref[...],
                        preferred_element_type=jnp.float32)
        z = jnp.dot(x2g, whh_ref[...], preferred_element_type=jnp.float32)
        z = z + jnp.dot(ft_ref[...], wf_ref[...],
                        preferred_element_type=jnp.float32)
        z = z + td_ref[...] * wt_ref[...] + const + b_ref[...]
        h = jnp.tanh(z)
        nrm = jnp.sqrt(jnp.sum(h * h, axis=1, keepdims=True))
        o_ref[...] = h / jnp.maximum(nrm, 1e-12)


def kernel(patient_dynamic, code_dynamic, init_code_dynamic, adj, patientid,
           codeid, ancestorid, features, timediffs, W1, W2, W_ih, b_ih, W_hh,
           b_hh):
    patient_row = jax.lax.dynamic_slice_in_dim(patient_dynamic, patientid, 1,
                                               axis=0)
    W_p_T = W_ih[:, :D].T
    w_t_row = W_ih[:, D:D + 1].T
    W_f_T = W_ih[:, D + 1:].T
    b = (b_ih + b_hh)[None, :]

    grid_spec = pltpu.PrefetchScalarGridSpec(
        num_scalar_prefetch=1,
        grid=(NA + NB,),
        in_specs=[
            pl.BlockSpec(memory_space=pltpu.MemorySpace.HBM),
            pl.BlockSpec(memory_space=pltpu.MemorySpace.HBM),
            pl.BlockSpec((N, D), lambda i, ids: (0, 0)),
            pl.BlockSpec((HR, D),
                         lambda i, ids: (jnp.minimum(i, NA - 1), 0)),
            pl.BlockSpec((HR, D),
                         lambda i, ids: (NA + jnp.minimum(i, NA - 1), 0)),
            pl.BlockSpec((D, D), lambda i, ids: (0, 0)),
            pl.BlockSpec((D, D), lambda i, ids: (0, 0)),
            pl.BlockSpec((GB, 1),
                         lambda i, ids: (jnp.maximum(i - NA, 0), 0)),
            pl.BlockSpec((GB, D),
                         lambda i, ids: (jnp.maximum(i - NA, 0), 0)),
            pl.BlockSpec((1, D), lambda i, ids: (0, 0)),
            pl.BlockSpec((D, D), lambda i, ids: (0, 0)),
            pl.BlockSpec((D, D), lambda i, ids: (0, 0)),
            pl.BlockSpec((1, D), lambda i, ids: (0, 0)),
            pl.BlockSpec((D, D), lambda i, ids: (0, 0)),
            pl.BlockSpec((1, D), lambda i, ids: (0, 0)),
        ],
        out_specs=pl.BlockSpec((GB, D),
                               lambda i, ids: (jnp.maximum(i - NA, 0), 0)),
        scratch_shapes=[
            pltpu.VMEM((NBUF, RB, N), jnp.float32),
            pltpu.VMEM((NBUF, GB, D), jnp.float32),
            pltpu.VMEM((N, 2 * D), jnp.float32),
            pltpu.SemaphoreType.DMA((NBUF,)),
            pltpu.SemaphoreType.DMA((NBUF,)),
        ],
    )
    return pl.pallas_call(
        _body,
        grid_spec=grid_spec,
        out_shape=jax.ShapeDtypeStruct((B, D), jnp.float32),
    )(codeid, adj, init_code_dynamic, code_dynamic, init_code_dynamic,
      init_code_dynamic, W1, W2, timediffs, features, patient_row, W_hh.T,
      W_f_T, w_t_row, W_p_T, b)


# speedup vs baseline: 1.0135x; 1.0135x over previous
import jax
import jax.numpy as jnp
from jax.experimental import pallas as pl
from jax.experimental.pallas import tpu as pltpu

N = 10000
D = 64
B = 2048
ALPHA = 0.5
RB = 400
NA = N // RB
GB = 256
NB = B // GB
NBUF = 3


def _body(id_ref, adj_ref, init_ref, x0_ref, iga_ref, w1_ref, w2_ref, td_ref,
          ft_ref, p_ref, whh_ref, wf_ref, wt_ref, wp_ref, b_ref, o_ref,
          a_buf, i_buf, s_buf, sem, isem):
    i = pl.program_id(0)
    nsteps = pl.num_programs(0)

    def issue(blk):
        slot = blk % NBUF

        @pl.when(blk < NA)
        def _():
            pltpu.make_async_copy(
                adj_ref.at[pl.ds(blk * RB, RB), :], a_buf.at[slot],
                sem.at[slot]).start()

        @pl.when(blk >= NA)
        def _():
            def body(j, _):
                r = id_ref[(blk - NA) * GB + j]
                pltpu.make_async_copy(
                    adj_ref.at[pl.ds(r, 1), :],
                    a_buf.at[slot, pl.ds(j, 1), :], sem.at[slot]).start()
                pltpu.make_async_copy(
                    init_ref.at[pl.ds(r, 1), :],
                    i_buf.at[slot, pl.ds(j, 1), :], isem.at[slot]).start()
                return 0
            jax.lax.fori_loop(0, GB, body, 0, unroll=8)

    @pl.when(i == 0)
    def _():
        issue(0)
        issue(1)

    @pl.when(i + 2 < nsteps)
    def _():
        issue(i + 2)

    slot = i % NBUF

    @pl.when(i == 0)
    def _():
        s_buf[:, :D] = jnp.dot(x0_ref[...], w1_ref[...],
                               preferred_element_type=jnp.float32)

    @pl.when(i < NA)
    def _():
        pltpu.make_async_copy(
            adj_ref.at[pl.ds(0, RB), :], a_buf.at[slot], sem.at[slot]).wait()

        y = jnp.dot(a_buf[slot], s_buf[:, :D],
                    preferred_element_type=jnp.float32)
        x1b = jnp.maximum(ALPHA * y + (1.0 - ALPHA) * iga_ref[...], 0.0)
        s_buf[pl.ds(i * RB, RB), D:] = jnp.dot(
            x1b, w2_ref[...], preferred_element_type=jnp.float32)

    @pl.when(i >= NA)
    def _():
        def wbody(j, _):
            pltpu.make_async_copy(
                adj_ref.at[pl.ds(0, 1), :], a_buf.at[slot, pl.ds(0, 1), :],
                sem.at[slot]).wait()
            pltpu.make_async_copy(
                init_ref.at[pl.ds(0, 1), :], i_buf.at[slot, pl.ds(0, 1), :],
                isem.at[slot]).wait()
            return 0
        jax.lax.fori_loop(0, GB, wbody, 0, unroll=8)

        y = jnp.dot(a_buf[slot, :GB, :], s_buf[:, D:],
                    preferred_element_type=jnp.float32)
        x2g = jnp.maximum(ALPHA * y + (1.0 - ALPHA) * i_buf[slot], 0.0)
        const = jnp.dot(p_ref[...], wp_ref[...],
                        preferred_element_type=jnp.float32)
        z = jnp.dot(x2g, whh_ref[...], preferred_element_type=jnp.float32)
        z = z + jnp.dot(ft_ref[...], wf_ref[...],
                        preferred_element_type=jnp.float32)
        z = z + td_ref[...] * wt_ref[...] + const + b_ref[...]
        h = jnp.tanh(z)
        nrm = jnp.sqrt(jnp.sum(h * h, axis=1, keepdims=True))
        o_ref[...] = h / jnp.maximum(nrm, 1e-12)


def kernel(patient_dynamic, code_dynamic, init_code_dynamic, adj, patientid,
           codeid, ancestorid, features, timediffs, W1, W2, W_ih, b_ih, W_hh,
           b_hh):
    patient_row = jax.lax.dynamic_slice_in_dim(patient_dynamic, patientid, 1,
                                               axis=0)
    W_p_T = W_ih[:, :D].T
    w_t_row = W_ih[:, D:D + 1].T
    W_f_T = W_ih[:, D + 1:].T
    b = (b_ih + b_hh)[None, :]

    grid_spec = pltpu.PrefetchScalarGridSpec(
        num_scalar_prefetch=1,
        grid=(NA + NB,),
        in_specs=[
            pl.BlockSpec(memory_space=pltpu.MemorySpace.HBM),
            pl.BlockSpec(memory_space=pltpu.MemorySpace.HBM),
            pl.BlockSpec((N, D), lambda i, ids: (0, 0)),
            pl.BlockSpec((RB, D),
                         lambda i, ids: (jnp.minimum(i, NA - 1), 0)),
            pl.BlockSpec((D, D), lambda i, ids: (0, 0)),
            pl.BlockSpec((D, D), lambda i, ids: (0, 0)),
            pl.BlockSpec((GB, 1),
                         lambda i, ids: (jnp.maximum(i - NA, 0), 0)),
            pl.BlockSpec((GB, D),
                         lambda i, ids: (jnp.maximum(i - NA, 0), 0)),
            pl.BlockSpec((1, D), lambda i, ids: (0, 0)),
            pl.BlockSpec((D, D), lambda i, ids: (0, 0)),
            pl.BlockSpec((D, D), lambda i, ids: (0, 0)),
            pl.BlockSpec((1, D), lambda i, ids: (0, 0)),
            pl.BlockSpec((D, D), lambda i, ids: (0, 0)),
            pl.BlockSpec((1, D), lambda i, ids: (0, 0)),
        ],
        out_specs=pl.BlockSpec((GB, D),
                               lambda i, ids: (jnp.maximum(i - NA, 0), 0)),
        scratch_shapes=[
            pltpu.VMEM((NBUF, RB, N), jnp.float32),
            pltpu.VMEM((NBUF, GB, D), jnp.float32),
            pltpu.VMEM((N, 2 * D), jnp.float32),
            pltpu.SemaphoreType.DMA((NBUF,)),
            pltpu.SemaphoreType.DMA((NBUF,)),
        ],
    )
    return pl.pallas_call(
        _body,
        grid_spec=grid_spec,
        out_shape=jax.ShapeDtypeStruct((B, D), jnp.float32),
    )(codeid, adj, init_code_dynamic, code_dynamic, init_code_dynamic, W1, W2,
      timediffs, features, patient_row, W_hh.T, W_f_T, w_t_row, W_p_T, b)


# speedup vs baseline: 1.0503x; 1.0364x over previous
import jax
import jax.numpy as jnp
from jax.experimental import pallas as pl
from jax.experimental.pallas import tpu as pltpu

N = 10000
D = 64
B = 2048
ALPHA = 0.5
RB = 400
NA = N // RB
GB = 256
NB = B // GB
NBUF = 3


def _body(id_ref, adj_ref, init_ref, x0_ref, iga_ref, w1_ref, w2_ref, td_ref,
          ft_ref, p_ref, whh_ref, wf_ref, wt_ref, wp_ref, b_ref, o_ref,
          a_buf, i_all, s_buf, sem, isem):
    i = pl.program_id(0)
    nsteps = pl.num_programs(0)

    def issue(blk):
        slot = blk % NBUF

        @pl.when(blk < NA)
        def _():
            pltpu.make_async_copy(
                adj_ref.at[pl.ds(blk * RB, RB), :], a_buf.at[slot],
                sem.at[slot]).start()

        @pl.when(blk >= NA)
        def _():
            def body(j, _):
                r = id_ref[(blk - NA) * GB + j]
                pltpu.make_async_copy(
                    adj_ref.at[pl.ds(r, 1), :],
                    a_buf.at[slot, pl.ds(j, 1), :], sem.at[slot]).start()
                return 0
            jax.lax.fori_loop(0, GB, body, 0, unroll=8)

    @pl.when(i == 0)
    def _():
        issue(0)
        issue(1)

    @pl.when(i + 2 < nsteps)
    def _():
        issue(i + 2)

    slot = i % NBUF

    @pl.when(i == 0)
    def _():
        s_buf[:, :D] = jnp.dot(x0_ref[...], w1_ref[...],
                               preferred_element_type=jnp.float32)

    @pl.when(i < B // 128)
    def _():
        def ibody(j, _):
            k = i * 128 + j
            r = id_ref[k]
            pltpu.make_async_copy(
                init_ref.at[pl.ds(r, 1), :], i_all.at[pl.ds(k, 1), :],
                isem).start()
            return 0
        jax.lax.fori_loop(0, 128, ibody, 0, unroll=8)

    @pl.when(i < NA)
    def _():
        pltpu.make_async_copy(
            adj_ref.at[pl.ds(0, RB), :], a_buf.at[slot], sem.at[slot]).wait()

        y = jnp.dot(a_buf[slot], s_buf[:, :D],
                    preferred_element_type=jnp.float32)
        x1b = jnp.maximum(ALPHA * y + (1.0 - ALPHA) * iga_ref[...], 0.0)
        s_buf[pl.ds(i * RB, RB), D:] = jnp.dot(
            x1b, w2_ref[...], preferred_element_type=jnp.float32)

    @pl.when(i >= NA)
    def _():
        @pl.when(i == NA)
        def _():
            pltpu.make_async_copy(
                init_ref.at[pl.ds(0, B), :], i_all, isem).wait()

        pltpu.make_async_copy(
            adj_ref.at[pl.ds(0, GB), :], a_buf.at[slot, pl.ds(0, GB), :],
            sem.at[slot]).wait()

        y = jnp.dot(a_buf[slot, :GB, :], s_buf[:, D:],
                    preferred_element_type=jnp.float32)
        ig = i_all[pl.ds((i - NA) * GB, GB), :]
        x2g = jnp.maximum(ALPHA * y + (1.0 - ALPHA) * ig, 0.0)
        const = jnp.dot(p_ref[...], wp_ref[...],
                        preferred_element_type=jnp.float32)
        z = jnp.dot(x2g, whh_ref[...], preferred_element_type=jnp.float32)
        z = z + jnp.dot(ft_ref[...], wf_ref[...],
                        preferred_element_type=jnp.float32)
        z = z + td_ref[...] * wt_ref[...] + const + b_ref[...]
        h = jnp.tanh(z)
        nrm = jnp.sqrt(jnp.sum(h * h, axis=1, keepdims=True))
        o_ref[...] = h / jnp.maximum(nrm, 1e-12)


def kernel(patient_dynamic, code_dynamic, init_code_dynamic, adj, patientid,
           codeid, ancestorid, features, timediffs, W1, W2, W_ih, b_ih, W_hh,
           b_hh):
    patient_row = jax.lax.dynamic_slice_in_dim(patient_dynamic, patientid, 1,
                                               axis=0)
    W_p_T = W_ih[:, :D].T
    w_t_row = W_ih[:, D:D + 1].T
    W_f_T = W_ih[:, D + 1:].T
    b = (b_ih + b_hh)[None, :]

    grid_spec = pltpu.PrefetchScalarGridSpec(
        num_scalar_prefetch=1,
        grid=(NA + NB,),
        in_specs=[
            pl.BlockSpec(memory_space=pltpu.MemorySpace.HBM),
            pl.BlockSpec(memory_space=pltpu.MemorySpace.HBM),
            pl.BlockSpec((N, D), lambda i, ids: (0, 0)),
            pl.BlockSpec((RB, D),
                         lambda i, ids: (jnp.minimum(i, NA - 1), 0)),
            pl.BlockSpec((D, D), lambda i, ids: (0, 0)),
            pl.BlockSpec((D, D), lambda i, ids: (0, 0)),
            pl.BlockSpec((GB, 1),
                         lambda i, ids: (jnp.maximum(i - NA, 0), 0)),
            pl.BlockSpec((GB, D),
                         lambda i, ids: (jnp.maximum(i - NA, 0), 0)),
            pl.BlockSpec((1, D), lambda i, ids: (0, 0)),
            pl.BlockSpec((D, D), lambda i, ids: (0, 0)),
            pl.BlockSpec((D, D), lambda i, ids: (0, 0)),
            pl.BlockSpec((1, D), lambda i, ids: (0, 0)),
            pl.BlockSpec((D, D), lambda i, ids: (0, 0)),
            pl.BlockSpec((1, D), lambda i, ids: (0, 0)),
        ],
        out_specs=pl.BlockSpec((GB, D),
                               lambda i, ids: (jnp.maximum(i - NA, 0), 0)),
        scratch_shapes=[
            pltpu.VMEM((NBUF, RB, N), jnp.float32),
            pltpu.VMEM((B, D), jnp.float32),
            pltpu.VMEM((N, 2 * D), jnp.float32),
            pltpu.SemaphoreType.DMA((NBUF,)),
            pltpu.SemaphoreType.DMA(()),
        ],
    )
    return pl.pallas_call(
        _body,
        grid_spec=grid_spec,
        out_shape=jax.ShapeDtypeStruct((B, D), jnp.float32),
    )(codeid, adj, init_code_dynamic, code_dynamic, init_code_dynamic, W1, W2,
      timediffs, features, patient_row, W_hh.T, W_f_T, w_t_row, W_p_T, b)
